# baseline (speedup 1.0000x reference)
import jax
import jax.numpy as jnp
from jax import lax
from jax.experimental import pallas as pl
from jax.experimental.pallas import tpu as pltpu
from jax.experimental.pallas import tpu_sc as plsc

B, H, S, D = 8, 16, 4096, 128
Q = 16
BH = B * H

NC, NS = 2, 16
NW = NC * NS
BH_PER_W = BH // NW

MBH = 64
SC_ROW0 = MBH * S
ROWS_PER_W = (BH - MBH) * S // NW

NBUF = 2
CH = 256
NCHUNK = ROWS_PER_W // CH


def _tc_k_body(pos_ref, kval_ref, kc_ref, ko_ref):
    ko_ref[...] = kc_ref[...]
    for q in range(Q):
        p = pos_ref[q]
        m = q
        for r in range(q + 1, Q):
            m = jnp.where(pos_ref[r] == p, r, m)
        ko_ref[0, pl.ds(p, 1), :] = kval_ref[0, pl.ds(m, 1), :]


def _tc_k(pos, kv, kc):
    cache_spec = pl.BlockSpec((1, S, D), lambda i: (i, 0, 0))
    val_spec = pl.BlockSpec((1, Q, D), lambda i: (i, 0, 0))
    return pl.pallas_call(
        _tc_k_body,
        grid=(BH,),
        in_specs=[pl.BlockSpec(memory_space=pltpu.SMEM), val_spec, cache_spec],
        out_specs=cache_spec,
        out_shape=jax.ShapeDtypeStruct((BH, S, D), jnp.float32),
        compiler_params=pltpu.CompilerParams(
            dimension_semantics=("arbitrary",),
        ),
    )(pos, kv, kc)


def _tc_v_prefix_body(vc_ref, vo_ref):
    vo_ref[...] = vc_ref[...]


def _tc_v_prefix(vc):
    cache_spec = pl.BlockSpec((1, S, D), lambda i: (i, 0, 0))
    return pl.pallas_call(
        _tc_v_prefix_body,
        grid=(MBH,),
        in_specs=[cache_spec],
        out_specs=cache_spec,
        out_shape=jax.ShapeDtypeStruct((BH, S, D), jnp.float32),
        compiler_params=pltpu.CompilerParams(
            dimension_semantics=("arbitrary",),
        ),
    )(vc)


def _sc_v_body(pos_hbm, vval_hbm, vc_hbm, vo_ref,
               pos_v, src_v, dst_v, vrows,
               b0, b1, si0, si1, so0, so1, sem):
    wid = lax.axis_index("s") * NC + lax.axis_index("c")
    row0 = SC_ROW0 + wid * ROWS_PER_W
    bufs = (b0, b1)
    sin = (si0, si1)
    sout = (so0, so1)

    for b in range(NBUF):
        pltpu.async_copy(vc_hbm.at[pl.ds(row0 + b * CH, CH)], bufs[b], sin[b])

    @pl.loop(0, NCHUNK, step=NBUF)
    def _(g):
        for b in range(NBUF):
            c = g + b
            pltpu.make_async_copy(vc_hbm.at[pl.ds(row0 + c * CH, CH)],
                                  bufs[b], sin[b]).wait()
            pltpu.async_copy(bufs[b], vo_ref.at[pl.ds(row0 + c * CH, CH)],
                             sout[b])
        for b in range(NBUF):
            c = g + b
            pltpu.make_async_copy(bufs[b],
                                  vo_ref.at[pl.ds(row0 + c * CH, CH)],
                                  sout[b]).wait()

            @pl.when(c + NBUF < NCHUNK)
            def _():
                pltpu.async_copy(vc_hbm.at[pl.ds(row0 + (c + NBUF) * CH, CH)],
                                 bufs[b], sin[b])

    pltpu.sync_copy(pos_hbm, pos_v)
    pos = pos_v[...]
    iota = lax.iota(jnp.int32, 16)
    m = iota
    for r in range(1, Q):
        pos_r = jnp.take_along_axis(pos, jnp.full((Q,), r, jnp.int32), axis=0)
        m = jnp.where(pos == pos_r, r, m)

    tc_share = MBH // NW
    sc_share = (BH - MBH) // NW
    bhs = ([tc_share * wid + j for j in range(tc_share)]
           + [MBH + sc_share * wid + j for j in range(sc_share)])
    for j, bh in enumerate(bhs):
        src_v[pl.ds(j * Q, Q)] = bh * Q + m
        dst_v[pl.ds(j * Q, Q)] = bh * S + pos

    pltpu.async_copy(vval_hbm.at[src_v], vrows, sem).wait()
    pltpu.async_copy(vrows, vo_ref.at[dst_v], sem).wait()


_sc_v = pl.kernel(
    _sc_v_body,
    out_type=(),
    mesh=plsc.VectorSubcoreMesh(core_axis_name="c", subcore_axis_name="s"),
    scratch_types=(
        [
            pltpu.VMEM((Q,), jnp.int32),
            pltpu.VMEM((BH_PER_W * Q,), jnp.int32),
            pltpu.VMEM((BH_PER_W * Q,), jnp.int32),
            pltpu.VMEM((BH_PER_W * Q, D), jnp.float32),
        ]
        + [pltpu.VMEM((CH, D), jnp.float32) for _ in range(NBUF)]
        + [pltpu.SemaphoreType.DMA for _ in range(2 * NBUF + 1)]
    ),
)


def kernel(input_pos, k_val, v_val, k_cache, v_cache):
    vo1 = _tc_v_prefix(v_cache.reshape(BH, S, D))
    vo_ref = jax.new_ref(vo1.reshape(BH * S, D))
    _sc_v(input_pos, v_val.reshape(BH * Q, D), v_cache.reshape(BH * S, D),
          vo_ref)
    ko = _tc_k(input_pos, k_val.reshape(BH, Q, D), k_cache.reshape(BH, S, D))
    return ko.reshape(B, H, S, D), vo_ref[...].reshape(B, H, S, D)


# speedup vs baseline: 1.1090x; 1.1090x over previous
import jax
import jax.numpy as jnp
from jax.experimental import pallas as pl
from jax.experimental.pallas import tpu as pltpu

B, H, S, D = 8, 16, 4096, 128
Q = 16
BH = B * H


def _body(pos_ref, kval_ref, vval_ref, kc_ref, vc_ref, ko_ref, vo_ref):
    ko_ref[...] = kc_ref[...]
    vo_ref[...] = vc_ref[...]
    for q in range(Q):
        p = pos_ref[q]
        m = q
        for r in range(q + 1, Q):
            m = jnp.where(pos_ref[r] == p, r, m)
        ko_ref[0, pl.ds(p, 1), :] = kval_ref[0, pl.ds(m, 1), :]
        vo_ref[0, pl.ds(p, 1), :] = vval_ref[0, pl.ds(m, 1), :]


def kernel(input_pos, k_val, v_val, k_cache, v_cache):
    kc = k_cache.reshape(BH, S, D)
    vc = v_cache.reshape(BH, S, D)
    kv = k_val.reshape(BH, Q, D)
    vv = v_val.reshape(BH, Q, D)
    cache_spec = pl.BlockSpec((1, S, D), lambda i: (i, 0, 0))
    val_spec = pl.BlockSpec((1, Q, D), lambda i: (i, 0, 0))
    ko, vo = pl.pallas_call(
        _body,
        grid=(BH,),
        in_specs=[
            pl.BlockSpec(memory_space=pltpu.SMEM),
            val_spec,
            val_spec,
            cache_spec,
            cache_spec,
        ],
        out_specs=[cache_spec, cache_spec],
        out_shape=[
            jax.ShapeDtypeStruct((BH, S, D), jnp.float32),
            jax.ShapeDtypeStruct((BH, S, D), jnp.float32),
        ],
        compiler_params=pltpu.CompilerParams(
            dimension_semantics=("arbitrary",),
        ),
    )(input_pos, kv, vv, kc, vc)
    return ko.reshape(B, H, S, D), vo.reshape(B, H, S, D)


# speedup vs baseline: 1.1243x; 1.0137x over previous
import jax
import jax.numpy as jnp
from jax.experimental import pallas as pl
from jax.experimental.pallas import tpu as pltpu

B, H, S, D = 8, 16, 4096, 128
Q = 16
BH = B * H
G = 2


def _body(pos_ref, kval_ref, vval_ref, kc_ref, vc_ref, ko_ref, vo_ref):
    ko_ref[...] = kc_ref[...]
    vo_ref[...] = vc_ref[...]
    for q in range(Q):
        p = pos_ref[q]
        m = q
        for r in range(q + 1, Q):
            m = jnp.where(pos_ref[r] == p, r, m)
        for g in range(G):
            ko_ref[g, pl.ds(p, 1), :] = kval_ref[g, pl.ds(m, 1), :]
            vo_ref[g, pl.ds(p, 1), :] = vval_ref[g, pl.ds(m, 1), :]


def kernel(input_pos, k_val, v_val, k_cache, v_cache):
    kc = k_cache.reshape(BH, S, D)
    vc = v_cache.reshape(BH, S, D)
    kv = k_val.reshape(BH, Q, D)
    vv = v_val.reshape(BH, Q, D)
    cache_spec = pl.BlockSpec((G, S, D), lambda i: (i, 0, 0))
    val_spec = pl.BlockSpec((G, Q, D), lambda i: (i, 0, 0))
    ko, vo = pl.pallas_call(
        _body,
        grid=(BH // G,),
        in_specs=[
            pl.BlockSpec(memory_space=pltpu.SMEM),
            val_spec,
            val_spec,
            cache_spec,
            cache_spec,
        ],
        out_specs=[cache_spec, cache_spec],
        out_shape=[
            jax.ShapeDtypeStruct((BH, S, D), jnp.float32),
            jax.ShapeDtypeStruct((BH, S, D), jnp.float32),
        ],
        compiler_params=pltpu.CompilerParams(
            dimension_semantics=("arbitrary",),
        ),
    )(input_pos, kv, vv, kc, vc)
    return ko.reshape(B, H, S, D), vo.reshape(B, H, S, D)
